# baseline (speedup 1.0000x reference)
import functools

import jax
import jax.numpy as jnp
from jax import lax
from jax.experimental import pallas as pl
from jax.experimental.pallas import tpu as pltpu
from jax.experimental.pallas import tpu_sc as plsc

B, C, D, H, W = 4, 64, 32, 32, 32
DHW = D * H * W
N = 50000
BN = B * N

_info = plsc.get_sparse_core_info()
NC = _info.num_cores
NS = _info.num_subcores
L = _info.num_lanes
NWORK = NC * NS

PTS_PER_TILE = -(-BN // NWORK)
PTS_PER_TILE = -(-PTS_PER_TILE // 16) * 16

U = 128
USHIFT = U.bit_length() - 1
NFULL = N // U
UI = -(-NFULL // NWORK)

_mesh = plsc.VectorSubcoreMesh(core_axis_name="c", subcore_axis_name="s")
_params = pltpu.CompilerParams(needs_layout_passes=False)


@functools.partial(
    pl.kernel,
    mesh=_mesh,
    compiler_params=_params,
    out_type=jax.ShapeDtypeStruct((BN,), jnp.int32),
    scratch_types=[
        pltpu.VMEM((3 * PTS_PER_TILE,), jnp.float32),
        pltpu.VMEM((PTS_PER_TILE,), jnp.int32),
    ],
)
def _lin_index_kernel(grid_hbm, lin_hbm, gbuf, lbuf):
    wid = lax.axis_index("s") * NC + lax.axis_index("c")
    base = jnp.where(wid == NWORK - 1, BN - PTS_PER_TILE,
                     wid * PTS_PER_TILE)
    for j in range(3):
        pltpu.sync_copy(grid_hbm.at[pl.ds(j * BN + base, PTS_PER_TILE)],
                        gbuf.at[pl.ds(j * PTS_PER_TILE, PTS_PER_TILE)])

    @plsc.parallel_loop(0, PTS_PER_TILE, 16, unroll=4)
    def body(o):
        x = gbuf[pl.ds(o, 16)].astype(jnp.int32)
        y = gbuf[pl.ds(PTS_PER_TILE + o, 16)].astype(jnp.int32)
        z = gbuf[pl.ds(2 * PTS_PER_TILE + o, 16)].astype(jnp.int32)
        lbuf[pl.ds(o, 16)] = (x * H + y) * W + z

    pltpu.sync_copy(lbuf, lin_hbm.at[pl.ds(base, PTS_PER_TILE)])


@functools.partial(
    pl.kernel,
    mesh=_mesh,
    compiler_params=_params,
    out_type=(jax.ShapeDtypeStruct((B, C, N), jnp.float32),
              jax.ShapeDtypeStruct((B, C, U), jnp.float32)),
    scratch_types=[
        pltpu.VMEM((U,), jnp.int32),
        pltpu.VMEM((U,), jnp.int32),
        pltpu.VMEM((U,), jnp.int32),
        pltpu.VMEM((U,), jnp.int32),
        pltpu.VMEM((U,), jnp.int32),
        pltpu.VMEM((U,), jnp.int32),
        pltpu.VMEM((U,), jnp.int32),
        pltpu.VMEM((U,), jnp.int32),
        pltpu.VMEM((U, 2 * C), jnp.float32),
        pltpu.VMEM((U, 2 * C), jnp.float32),
        pltpu.VMEM((U, 2 * C), jnp.float32),
        pltpu.VMEM((U, 2 * C), jnp.float32),
        pltpu.VMEM((C, U), jnp.float32),
        pltpu.VMEM((C, U), jnp.float32),
        pltpu.SemaphoreType.DMA,
        pltpu.SemaphoreType.DMA,
        pltpu.SemaphoreType.DMA,
        pltpu.SemaphoreType.DMA,
        pltpu.SemaphoreType.DMA,
        pltpu.SemaphoreType.DMA,
        pltpu.SemaphoreType.DMA,
        pltpu.SemaphoreType.DMA,
        pltpu.SemaphoreType.DMA,
        pltpu.SemaphoreType.DMA,
    ],
)
def _gather_kernel(table_hbm, lin_hbm, out_hbm, tail_hbm,
                   li0, li1, li2, li3, gi0, gi1, gi2, gi3,
                   r0, r1, r2, r3, t0, t1,
                   sli0, sli1, sli2, sli3, sr0, sr1, sr2, sr3, so0, so1):
    wid = lax.axis_index("s") * NC + lax.axis_index("c")
    libufs, lisems = (li0, li1, li2, li3), (sli0, sli1, sli2, sli3)
    gibufs = (gi0, gi1, gi2, gi3)
    rbufs, rsems = (r0, r1, r2, r3), (sr0, sr1, sr2, sr3)
    tbufs, tsems = (t0, t1), (so0, so1)
    iotav = lax.iota(jnp.int32, 16)

    def unit_tc(i):
        tc = wid + NWORK * i
        return jnp.where(tc >= NFULL, wid, tc)

    def idx_load(b, i, s):
        off = pl.multiple_of(b * N + unit_tc(i) * U, 16)
        return pltpu.async_copy(lin_hbm.at[pl.ds(off, U)],
                                libufs[s], lisems[s])

    def make_gidx(b, s):
        src, dst = libufs[s], gibufs[s]

        @plsc.parallel_loop(0, U, 16, unroll=4)
        def body(o, src=src, dst=dst):
            dst[pl.ds(o, 16)] = src[pl.ds(o, 16)] + b * DHW

    def rows_gather(s):
        return pltpu.async_copy(table_hbm.at[gibufs[s]], rbufs[s], rsems[s])

    def transpose_rt(s, ts):
        rbuf, tbuf = rbufs[s], tbufs[ts]

        @plsc.parallel_loop(0, C * U, 16, unroll=8)
        def body(o, rbuf=rbuf, tbuf=tbuf):
            c = o >> USHIFT
            colbase = o & (U - 1)
            rv = iotav + colbase
            cv = jnp.full((16,), 0, jnp.int32) + c
            tbuf[c, pl.ds(colbase, 16)] = plsc.load_gather(rbuf, [rv, cv])

    def out_store(b, i, ts):
        off = pl.multiple_of(unit_tc(i) * U, U)
        return pltpu.async_copy(tbufs[ts],
                                out_hbm.at[b, :, pl.ds(off, U)],
                                tsems[ts])

    DEPTH = 4

    def batch_body(b, carry):
        out_pending = [None, None]
        idx_pending = [idx_load(b, u, u % DEPTH) if u < UI else None
                       for u in range(DEPTH)]
        rows_pending = [None] * DEPTH
        for u in range(min(2, UI)):
            idx_pending[u].wait()
            idx_pending[u] = None
            make_gidx(b, u % DEPTH)
            rows_pending[u] = rows_gather(u % DEPTH)
        for i in range(UI):
            s = i % DEPTH
            ts = i % 2
            u2 = i + 2
            if u2 < UI:
                s2 = u2 % DEPTH
                idx_pending[s2].wait()
                idx_pending[s2] = None
                make_gidx(b, s2)
                rows_pending[s2] = rows_gather(s2)
            u4 = i + DEPTH
            if u4 < UI:
                idx_pending[u4 % DEPTH] = idx_load(b, u4, u4 % DEPTH)
            rows_pending[s].wait()
            rows_pending[s] = None
            if out_pending[ts] is not None:
                out_pending[ts].wait()
            transpose_rt(s, ts)
            out_pending[ts] = out_store(b, i, ts)
        for ts in range(2):
            if out_pending[ts] is not None:
                out_pending[ts].wait()
        return carry

    lax.fori_loop(0, B, batch_body, 0)

    def tail_body(b, carry):
        @pl.when(wid == b)
        def _():
            pltpu.sync_copy(lin_hbm.at[pl.ds(b * N + N - U, U)], li0)
            make_gidx(b, 0)
            pltpu.async_copy(table_hbm.at[gi0], r0, sr0).wait()
            transpose_rt(0, 0)
            pltpu.async_copy(t0, tail_hbm.at[b], so0).wait()
        return carry

    lax.fori_loop(0, B, tail_body, 0)


_TAIL = N % U


def kernel(input_feats, sampling_grid):
    table = jnp.transpose(input_feats, (0, 2, 3, 4, 1)).reshape(B * DHW, C)
    table = jnp.pad(table, ((0, 0), (0, C)))
    gt = jnp.transpose(sampling_grid, (2, 0, 1)).reshape(3 * BN)
    lin = _lin_index_kernel(gt)
    out_main, tail = _gather_kernel(table, lin)
    return lax.dynamic_update_slice(out_main, tail[:, :, U - _TAIL:],
                                    (0, 0, N - _TAIL))
